# baseline (speedup 1.0000x reference)
import functools

import jax
import jax.numpy as jnp
from jax import lax
from jax.experimental import pallas as pl
from jax.experimental.pallas import tpu as pltpu

_BN_EPS = 1e-3
_CHUNKS0 = 4
_CHUNKS1 = 8


def _fused_kernel(s_ref, w_ref, g_ref, b_ref, x_ref, o_ref,
                  xs_ref, gram_ref, s1_ref, wb_ref, bias_ref,
                  *, nt, inv_count):
    ph = pl.program_id(0)
    ni = pl.program_id(1)
    ti = pl.program_id(2)
    blk = ni * nt + ti

    @pl.when(ph == 0)
    def _stats():
        @pl.when(blk == 0)
        def _():
            gram_ref[...] = jnp.zeros_like(gram_ref)
            s1_ref[...] = jnp.zeros_like(s1_ref)

        c, bh, w = x_ref.shape
        rows = bh // _CHUNKS0
        lanes = rows * w

        def _slab0(j, _):
            h0 = pl.multiple_of(j * rows, rows)
            xf = x_ref[:, pl.ds(h0, rows), :]
            xb = xf.astype(jnp.bfloat16).reshape(c, lanes)
            l0 = pl.multiple_of(j * lanes, lanes)
            xs_ref[blk, :, pl.ds(l0, lanes)] = xb
            gram_ref[...] += lax.dot_general(
                xb, xb, (((1,), (1,)), ((), ())),
                preferred_element_type=jnp.float32)
            s1_ref[...] += jnp.sum(xb, axis=1, keepdims=True,
                                   dtype=jnp.float32)
            return 0

        lax.fori_loop(0, _CHUNKS0, _slab0, 0)

    @pl.when(ph == 1)
    def _apply():
        @pl.when(blk == 0)
        def _():
            hs = jnp.clip(s_ref[...] * (1.0 / 6.0) + 0.5, 0.0, 1.0)
            wp = w_ref[...] * hs
            mean = jnp.dot(wp, s1_ref[...],
                           preferred_element_type=jnp.float32) * inv_count
            gw = jnp.dot(wp, gram_ref[...],
                         preferred_element_type=jnp.float32)
            ey2 = jnp.sum(gw * wp, axis=1, keepdims=True) * inv_count
            var = ey2 - mean * mean
            inv_std = lax.rsqrt(var + _BN_EPS)
            scale = g_ref[...] * inv_std
            bias_ref[...] = (b_ref[...] - mean * scale)[:, :, None]
            wb_ref[...] = (wp * scale).astype(jnp.bfloat16)

        co, bh, w = o_ref.shape
        rows = bh // _CHUNKS1
        lanes = rows * w

        def _slab1(j, _):
            l0 = pl.multiple_of(j * lanes, lanes)
            y = jnp.dot(wb_ref[...], xs_ref[blk, :, pl.ds(l0, lanes)],
                        preferred_element_type=jnp.float32)
            h0 = pl.multiple_of(j * rows, rows)
            o_ref[:, pl.ds(h0, rows), :] = (
                y.reshape(co, rows, w) + bias_ref[...])
            return 0

        lax.fori_loop(0, _CHUNKS1, _slab1, 0)


def kernel(x10, x6, weight, gamma, beta):
    n, c, h, w_sp = x6.shape
    c_out = weight.shape[0]

    bh = h
    nt = h // bh

    s = x10.reshape(1, c).astype(jnp.float32)
    w2d = weight.reshape(c_out, c).astype(jnp.float32)
    g = gamma.reshape(c_out, 1).astype(jnp.float32)
    b = beta.reshape(c_out, 1).astype(jnp.float32)

    def small(shape):
        return pl.BlockSpec(shape, lambda ph, ni, ti: (0, 0))

    out = pl.pallas_call(
        functools.partial(_fused_kernel, nt=nt,
                          inv_count=1.0 / float(n * h * w_sp)),
        out_shape=jax.ShapeDtypeStruct((n, c_out, h, w_sp), jnp.float32),
        grid=(2, n, nt),
        in_specs=[
            small((1, c)),
            small((c_out, c)),
            small((c_out, 1)),
            small((c_out, 1)),
            pl.BlockSpec((pl.Squeezed(), c, bh, w_sp),
                         lambda ph, ni, ti: (ni * (1 - ph), 0, ti * (1 - ph), 0)),
        ],
        out_specs=pl.BlockSpec((pl.Squeezed(), c_out, bh, w_sp),
                               lambda ph, ni, ti: (ni * ph, 0, ti * ph, 0)),
        scratch_shapes=[
            pltpu.VMEM((n * nt, c, bh * w_sp), jnp.bfloat16),
            pltpu.VMEM((c, c), jnp.float32),
            pltpu.VMEM((c, 1), jnp.float32),
            pltpu.VMEM((c_out, c), jnp.bfloat16),
            pltpu.VMEM((c_out, 1, 1), jnp.float32),
        ],
        compiler_params=pltpu.CompilerParams(
            dimension_semantics=("arbitrary", "arbitrary", "arbitrary"),
            vmem_limit_bytes=60 * 1024 * 1024),
    )(s, w2d, g, b, x6)

    return out


# speedup vs baseline: 1.0838x; 1.0838x over previous
import functools

import jax
import jax.numpy as jnp
from jax import lax
from jax.experimental import pallas as pl
from jax.experimental.pallas import tpu as pltpu

_BN_EPS = 1e-3
_CHUNKS0 = 4
_CHUNKS1 = 4


def _fused_kernel(s_ref, w_ref, g_ref, b_ref, x_ref, o_ref,
                  xs_ref, gram_ref, s1_ref, wb_ref, bias_ref,
                  *, nt, inv_count):
    ph = pl.program_id(0)
    ni = pl.program_id(1)
    ti = pl.program_id(2)
    blk = ni * nt + ti

    @pl.when(jnp.logical_and(ph == 0, blk == 0))
    def _init():
        gram_ref[...] = jnp.zeros_like(gram_ref)
        s1_ref[...] = jnp.zeros_like(s1_ref)

    c, bh, w = x_ref.shape
    rows0 = bh // _CHUNKS0
    lanes0 = rows0 * w

    def _slab0(j, _):
        h0 = pl.multiple_of(j * rows0, rows0)
        xf = x_ref[:, pl.ds(h0, rows0), :]
        xb = xf.astype(jnp.bfloat16).reshape(c, lanes0)
        l0 = pl.multiple_of(j * lanes0, lanes0)
        xs_ref[blk, :, pl.ds(l0, lanes0)] = xb
        gram_ref[...] += lax.dot_general(
            xb, xb, (((1,), (1,)), ((), ())),
            preferred_element_type=jnp.float32)
        s1_ref[...] += jnp.sum(xb, axis=1, keepdims=True,
                               dtype=jnp.float32)
        return 0

    lax.fori_loop(0, (1 - ph) * _CHUNKS0, _slab0, 0)

    @pl.when(jnp.logical_and(ph == 1, blk == 0))
    def _fold():
        hs = jnp.clip(s_ref[...] * (1.0 / 6.0) + 0.5, 0.0, 1.0)
        wp = w_ref[...] * hs
        mean = jnp.dot(wp, s1_ref[...],
                       preferred_element_type=jnp.float32) * inv_count
        gw = jnp.dot(wp, gram_ref[...],
                     preferred_element_type=jnp.float32)
        ey2 = jnp.sum(gw * wp, axis=1, keepdims=True) * inv_count
        var = ey2 - mean * mean
        inv_std = lax.rsqrt(var + _BN_EPS)
        scale = g_ref[...] * inv_std
        bias_ref[...] = (b_ref[...] - mean * scale)[:, :, None]
        wb_ref[...] = (wp * scale).astype(jnp.bfloat16)

    co = o_ref.shape[0]
    rows1 = bh // _CHUNKS1
    lanes1 = rows1 * w

    def _slab1(j, _):
        l0 = pl.multiple_of(j * lanes1, lanes1)
        y = jnp.dot(wb_ref[...], xs_ref[blk, :, pl.ds(l0, lanes1)],
                    preferred_element_type=jnp.float32)
        h0 = pl.multiple_of(j * rows1, rows1)
        o_ref[:, pl.ds(h0, rows1), :] = (
            y.reshape(co, rows1, w) + bias_ref[...])
        return 0

    lax.fori_loop(0, ph * _CHUNKS1, _slab1, 0)


def kernel(x10, x6, weight, gamma, beta):
    n, c, h, w_sp = x6.shape
    c_out = weight.shape[0]

    bh = h
    nt = h // bh

    s = x10.reshape(1, c).astype(jnp.float32)
    w2d = weight.reshape(c_out, c).astype(jnp.float32)
    g = gamma.reshape(c_out, 1).astype(jnp.float32)
    b = beta.reshape(c_out, 1).astype(jnp.float32)

    def small(shape):
        return pl.BlockSpec(shape, lambda ph, ni, ti: (0, 0))

    out = pl.pallas_call(
        functools.partial(_fused_kernel, nt=nt,
                          inv_count=1.0 / float(n * h * w_sp)),
        out_shape=jax.ShapeDtypeStruct((n, c_out, h, w_sp), jnp.float32),
        grid=(2, n, nt),
        in_specs=[
            small((1, c)),
            small((c_out, c)),
            small((c_out, 1)),
            small((c_out, 1)),
            pl.BlockSpec((pl.Squeezed(), c, bh, w_sp),
                         lambda ph, ni, ti: (ni * (1 - ph), 0, ti * (1 - ph), 0)),
        ],
        out_specs=pl.BlockSpec((pl.Squeezed(), c_out, bh, w_sp),
                               lambda ph, ni, ti: (ni * ph, 0, ti * ph, 0)),
        scratch_shapes=[
            pltpu.VMEM((n * nt, c, bh * w_sp), jnp.bfloat16),
            pltpu.VMEM((c, c), jnp.float32),
            pltpu.VMEM((c, 1), jnp.float32),
            pltpu.VMEM((c_out, c), jnp.bfloat16),
            pltpu.VMEM((c_out, 1, 1), jnp.float32),
        ],
        compiler_params=pltpu.CompilerParams(
            dimension_semantics=("arbitrary", "arbitrary", "arbitrary"),
            vmem_limit_bytes=60 * 1024 * 1024),
    )(s, w2d, g, b, x6)

    return out


# speedup vs baseline: 1.1845x; 1.0929x over previous
import functools

import jax
import jax.numpy as jnp
from jax import lax
from jax.experimental import pallas as pl
from jax.experimental.pallas import tpu as pltpu

_BN_EPS = 1e-3


def _fused_kernel(s_ref, w_ref, g_ref, b_ref, x_ref, o_ref,
                  xs_ref, gram_ref, s1_ref, wb_ref, bias_ref,
                  *, nt, inv_count):
    ph = pl.program_id(0)
    ni = pl.program_id(1)
    ti = pl.program_id(2)
    blk = ni * nt + ti

    @pl.when(ph == 0)
    def _stats():
        @pl.when(blk == 0)
        def _():
            gram_ref[...] = jnp.zeros_like(gram_ref)
            s1_ref[...] = jnp.zeros_like(s1_ref)

        xf3 = x_ref[...]
        c = xf3.shape[0]
        xb = xf3.astype(jnp.bfloat16).reshape(c, -1)
        xs_ref[blk] = xb
        gram_ref[...] += lax.dot_general(
            xb, xb, (((1,), (1,)), ((), ())),
            preferred_element_type=jnp.float32)
        s1_ref[...] += jnp.sum(xb, axis=1, keepdims=True,
                               dtype=jnp.float32)

    @pl.when(ph == 1)
    def _apply():
        @pl.when(blk == 0)
        def _():
            hs = jnp.clip(s_ref[...] * (1.0 / 6.0) + 0.5, 0.0, 1.0)
            wp = w_ref[...] * hs
            mean = jnp.dot(wp, s1_ref[...],
                           preferred_element_type=jnp.float32) * inv_count
            gw = jnp.dot(wp, gram_ref[...],
                         preferred_element_type=jnp.float32)
            ey2 = jnp.sum(gw * wp, axis=1, keepdims=True) * inv_count
            var = ey2 - mean * mean
            inv_std = lax.rsqrt(var + _BN_EPS)
            scale = g_ref[...] * inv_std
            bias_ref[...] = (b_ref[...] - mean * scale)[:, :, None]
            wb_ref[...] = (wp * scale).astype(jnp.bfloat16)

        y = jnp.dot(wb_ref[...], xs_ref[blk],
                    preferred_element_type=jnp.float32)
        co, bh, w = o_ref.shape
        o_ref[...] = y.reshape(co, bh, w) + bias_ref[...]


def kernel(x10, x6, weight, gamma, beta):
    n, c, h, w_sp = x6.shape
    c_out = weight.shape[0]

    bh = h
    nt = h // bh

    s = x10.reshape(1, c).astype(jnp.float32)
    w2d = weight.reshape(c_out, c).astype(jnp.float32)
    g = gamma.reshape(c_out, 1).astype(jnp.float32)
    b = beta.reshape(c_out, 1).astype(jnp.float32)

    def small(shape):
        return pl.BlockSpec(shape, lambda ph, ni, ti: (0, 0))

    out = pl.pallas_call(
        functools.partial(_fused_kernel, nt=nt,
                          inv_count=1.0 / float(n * h * w_sp)),
        out_shape=jax.ShapeDtypeStruct((n, c_out, h, w_sp), jnp.float32),
        grid=(2, n, nt),
        in_specs=[
            small((1, c)),
            small((c_out, c)),
            small((c_out, 1)),
            small((c_out, 1)),
            pl.BlockSpec((pl.Squeezed(), c, bh, w_sp),
                         lambda ph, ni, ti: (ni * (1 - ph), 0, ti * (1 - ph), 0)),
        ],
        out_specs=pl.BlockSpec((pl.Squeezed(), c_out, bh, w_sp),
                               lambda ph, ni, ti: (ni * ph, 0, ti * ph, 0)),
        scratch_shapes=[
            pltpu.VMEM((n * nt, c, bh * w_sp), jnp.bfloat16),
            pltpu.VMEM((c, c), jnp.float32),
            pltpu.VMEM((c, 1), jnp.float32),
            pltpu.VMEM((c_out, c), jnp.bfloat16),
            pltpu.VMEM((c_out, 1, 1), jnp.float32),
        ],
        compiler_params=pltpu.CompilerParams(
            dimension_semantics=("arbitrary", "arbitrary", "arbitrary"),
            vmem_limit_bytes=60 * 1024 * 1024),
    )(s, w2d, g, b, x6)

    return out
